# baseline (speedup 1.0000x reference)
import functools

import jax
import jax.numpy as jnp
from jax import lax
from jax.experimental import pallas as pl
from jax.experimental.pallas import tpu as pltpu
from jax.experimental.pallas import tpu_sc as plsc

NUM_LOCATION = 100000
EMBED_DIM = 128
BATCH = 16384
N_FEA = 26

NC = 2
NS = 16
NW = NC * NS
B_PER_W = BATCH // NW
L = 16


def _make_gather():
  mesh = plsc.VectorSubcoreMesh(core_axis_name="c", subcore_axis_name="s")

  @functools.partial(
      pl.kernel,
      out_type=jax.ShapeDtypeStruct((BATCH, EMBED_DIM), jnp.float32),
      mesh=mesh,
      scratch_types=[
          pltpu.VMEM((B_PER_W,), jnp.int32),
          pltpu.VMEM((B_PER_W,), jnp.int32),
          pltpu.VMEM((B_PER_W, EMBED_DIM), jnp.float32),
          pltpu.SemaphoreType.DMA,
      ],
  )
  def gather_kernel(fea_hbm, table_hbm, out_hbm, pos_v, idx_v, rows_v, sem):
    wid = lax.axis_index("s") * NC + lax.axis_index("c")
    base = wid * B_PER_W
    for j in range(B_PER_W // L):
      rows = lax.iota(jnp.int32, L) + (base + j * L)
      pos_v[pl.ds(j * L, L)] = rows * N_FEA
    pltpu.async_copy(fea_hbm.at[pos_v], idx_v, sem).wait()
    pltpu.async_copy(table_hbm.at[idx_v], rows_v, sem).wait()
    pltpu.sync_copy(rows_v, out_hbm.at[pl.ds(base, B_PER_W)])

  return gather_kernel


_gather = _make_gather()


@jax.jit
def kernel(user_fea, embedding_location):
  fea_flat = user_fea.reshape(-1).astype(jnp.int32)
  return _gather(fea_flat, embedding_location)
